# baseline (speedup 1.0000x reference)
import functools

import jax
import jax.numpy as jnp
from jax import lax
from jax.experimental import pallas as pl
from jax.experimental.pallas import tpu as pltpu
from jax.experimental.pallas import tpu_sc as plsc

_ALPHA = 0.6
M_S = 4
K = 16
NUM_WORKERS = 32


def _dense_body(s_ref, q4_ref, df_ref, s0_ref):
    s = s_ref[...]
    p = jax.nn.sigmoid(s)
    ce0 = jnp.maximum(s, 0.0) + jnp.log1p(jnp.exp(-jnp.abs(s)))
    loss0 = (1.0 - _ALPHA) * ce0 * p * p
    one_m_p = 1.0 - p
    loss1 = _ALPHA * (ce0 - s) * one_m_p * one_m_p
    q4_ref[...] = jnp.sqrt(jnp.sqrt(jnp.maximum(one_m_p, 0.0)))
    df_ref[...] = loss1 - loss0
    @pl.when(pl.program_id(0) == 0)
    def _():
        s0_ref[0, 0] = 0.0

    s0_ref[0, 0] += jnp.sum(loss0)


def _dense_stage(s2d, grid):
    rows = s2d.shape[0] // grid
    q4, df, s0p = pl.pallas_call(
        _dense_body,
        grid=(grid,),
        in_specs=[
            pl.BlockSpec((rows, 128), lambda i: (i, 0)),
        ],
        out_specs=[
            pl.BlockSpec((rows, 128), lambda i: (i, 0)),
            pl.BlockSpec((rows, 128), lambda i: (i, 0)),
            pl.BlockSpec((1, 1), lambda i: (0, 0), memory_space=pltpu.SMEM),
        ],
        out_shape=[
            jax.ShapeDtypeStruct(s2d.shape, jnp.float32),
            jax.ShapeDtypeStruct(s2d.shape, jnp.float32),
            jax.ShapeDtypeStruct((1, 1), jnp.float32),
        ],
    )(s2d)
    return q4, df, s0p


def _make_greedy_kernel(n_blocks, w_s, tb_rows):
    nrows5 = n_blocks * K // 128
    octets = nrows5 // 8
    big_o = -(-octets // NUM_WORKERS)
    n_big = octets - (big_o - 1) * NUM_WORKERS
    rows = big_o * 8
    trows = 64
    mesh = plsc.VectorSubcoreMesh(core_axis_name="c", subcore_axis_name="s")

    goff = [(i - (M_S - 1) / 2.0) / float(M_S) for i in range(M_S)]

    @functools.partial(
        pl.kernel,
        mesh=mesh,
        out_type=jax.ShapeDtypeStruct((NUM_WORKERS, 2 * K), jnp.float32),
        scratch_types=[
            pltpu.VMEM((rows, 128), jnp.float32),
            pltpu.VMEM((rows, 128), jnp.float32),
            pltpu.VMEM((rows, 128), jnp.float32),
            pltpu.VMEM((rows, 128), jnp.float32),
            pltpu.VMEM((64 * M_S * w_s + K,), jnp.float32),
            pltpu.VMEM((2 * K,), jnp.float32),
            pltpu.VMEM((K,), jnp.float32),
            pltpu.VMEM((K,), jnp.float32),
        ],
    )
    def greedy(q4_hbm, df_hbm, oy_hbm, ox_hbm, tb_hbm, out_hbm,
               q4_v, df_v, oy_v, ox_v, tb_v, acc_v, used_v, bc_v):
        wid = lax.axis_index("s") * 2 + lax.axis_index("c")
        o0 = big_o * wid - jnp.maximum(wid - n_big, 0)
        nblk = 64 * (big_o - jnp.minimum(jnp.maximum(wid - n_big + 1, 0), 1))
        srow = 8 * o0
        srow_cl = jnp.minimum(srow, nrows5 - rows)
        rowoff = srow - srow_cl
        g0 = 64 * o0
        lo = M_S * (g0 // w_s)
        start8 = jnp.minimum(8 * (lo // 8), tb_rows - trows)
        pltpu.sync_copy(q4_hbm.at[pl.ds(srow_cl, rows)], q4_v)
        pltpu.sync_copy(df_hbm.at[pl.ds(srow_cl, rows)], df_v)
        pltpu.sync_copy(oy_hbm.at[pl.ds(srow_cl, rows)], oy_v)
        pltpu.sync_copy(ox_hbm.at[pl.ds(srow_cl, rows)], ox_v)
        w_full = M_S * w_s
        pltpu.sync_copy(tb_hbm.at[pl.ds(start8 * w_full, trows * w_full)],
                        tb_v.at[pl.ds(0, trows * w_full)])

        iota = lax.iota(jnp.int32, K)
        iota_f = iota.astype(jnp.float32)
        huge = jnp.float32(1e30)
        xor_idx = [jnp.bitwise_xor(iota, jnp.int32(s)) for s in (1, 2, 4, 8)]

        def _perm(x, idx):
            return lax.gather(
                x, idx[:, None],
                dimension_numbers=lax.GatherDimensionNumbers(
                    offset_dims=(), collapsed_slice_dims=(0,),
                    start_index_map=(0,)),
                slice_sizes=(1,),
                mode=lax.GatherScatterMode.PROMISE_IN_BOUNDS)

        def _allmin(x):
            for idx in xor_idx:
                x = jnp.minimum(x, _perm(x, idx))
            return x

        def one_block(b):
            r = rowoff + b // 8
            c = (b % 8) * K
            qv = q4_v[r, pl.ds(c, K)]
            dfv = df_v[r, pl.ds(c, K)]
            oyv = oy_v[r, pl.ds(c, K)]
            oxv = ox_v[r, pl.ds(c, K)]
            g = g0 + b
            ws = g % w_s
            rb = M_S * (g // w_s) - start8
            base8 = rb * w_full + 8 * (ws // 2)
            o4 = (ws % 2).astype(jnp.float32)
            o4c = 1.0 - o4
            trow = [tb_v[pl.ds(base8 + w_full * jy, K)] for jy in range(M_S)]
            tj = [trow[j // M_S][j % M_S] * o4c
                  + trow[j // M_S][4 + (j % M_S)] * o4
                  for j in range(K)]
            dys = [jnp.abs(oyv - g_) for g_ in goff]
            dxs = [jnp.abs(oxv - g_) for g_ in goff]
            dyqs = [dy * qv for dy in dys]
            dxqs = [dx * qv for dx in dxs]
            used_v[...] = jnp.zeros((K,), jnp.float32)
            bc_v[...] = jnp.zeros((K,), jnp.float32)
            ts = jnp.float32(0.0)
            for j in range(K):
                jy, jx = j // M_S, j % M_S

                @pl.when(tj[j] > 0.0)
                def _(jy=jy, jx=jx):
                    used = used_v[...]
                    keyv = dyqs[jy] + dxqs[jx] + used * huge
                    minv = _allmin(keyv)
                    sel = 1.0 - jnp.sign(keyv - minv)
                    used_v[...] = used + sel
                    bc_v[...] = bc_v[...] + sel * (
                        dfv + 10.0 * (dys[jy] + dxs[jx]))

                ts = ts + tj[j]
            return bc_v[...], ts

        def block_body(i, carry):
            accv, nps = carry
            bc, ts = one_block(i)
            return accv + bc, nps + ts

        accv, nps = lax.fori_loop(
            0, nblk, block_body,
            (jnp.zeros((K,), jnp.float32), jnp.float32(0.0)))
        acc_v[pl.ds(0, K)] = accv
        acc_v[pl.ds(K, K)] = (1.0 - jnp.abs(jnp.sign(iota_f))) * nps
        pltpu.sync_copy(acc_v, out_hbm.at[wid])

    return greedy


def kernel(pred_scores, pred_offsets, H, W, M, target):
    B, HW, _ = pred_scores.shape
    n_blocks = B * HW
    h_s = target.shape[1] // M_S
    w_s = target.shape[2] // M_S

    tb = (target > 0).astype(jnp.float32).reshape(-1)
    s = pred_scores.reshape(n_blocks, K)
    oy = pred_offsets[..., 0].reshape(n_blocks, K)
    ox = pred_offsets[..., 1].reshape(n_blocks, K)

    ncols = (n_blocks * K) // 128
    grid = 5
    q4, df, s0p = _dense_stage(s.reshape(ncols, 128), grid)

    partials = _make_greedy_kernel(n_blocks, w_s, B * target.shape[1])(
        q4, df, oy.reshape(ncols, 128), ox.reshape(ncols, 128), tb)

    npos = jnp.maximum(partials[:, K:].sum(), 1.0)
    return (s0p.sum() + partials[:, :K].sum()) / npos


# speedup vs baseline: 2.0517x; 2.0517x over previous
import functools

import jax
import jax.numpy as jnp
from jax import lax
from jax.experimental import pallas as pl
from jax.experimental.pallas import tpu as pltpu
from jax.experimental.pallas import tpu_sc as plsc

_ALPHA = 0.6
M_S = 4
K = 16
NUM_WORKERS = 32


def _dense_body(s_ref, q4_ref, df_ref, s0_ref):
    s = s_ref[...]
    p = jax.nn.sigmoid(s)
    ce0 = jnp.maximum(s, 0.0) + jnp.log1p(jnp.exp(-jnp.abs(s)))
    loss0 = (1.0 - _ALPHA) * ce0 * p * p
    one_m_p = 1.0 - p
    loss1 = _ALPHA * (ce0 - s) * one_m_p * one_m_p
    q4_ref[...] = jnp.sqrt(jnp.sqrt(jnp.maximum(one_m_p, 0.0)))
    df_ref[...] = loss1 - loss0
    @pl.when(pl.program_id(0) == 0)
    def _():
        s0_ref[0, 0] = 0.0

    s0_ref[0, 0] += jnp.sum(loss0)


def _dense_stage(s2d, grid):
    rows = s2d.shape[0] // grid
    q4, df, s0p = pl.pallas_call(
        _dense_body,
        grid=(grid,),
        in_specs=[
            pl.BlockSpec((rows, 128), lambda i: (i, 0)),
        ],
        out_specs=[
            pl.BlockSpec((rows, 128), lambda i: (i, 0)),
            pl.BlockSpec((rows, 128), lambda i: (i, 0)),
            pl.BlockSpec((1, 1), lambda i: (0, 0), memory_space=pltpu.SMEM),
        ],
        out_shape=[
            jax.ShapeDtypeStruct(s2d.shape, jnp.float32),
            jax.ShapeDtypeStruct(s2d.shape, jnp.float32),
            jax.ShapeDtypeStruct((1, 1), jnp.float32),
        ],
    )(s2d)
    return q4, df, s0p


def _make_greedy_kernel(n_blocks, w_s, tb_rows):
    nrows5 = n_blocks * K // 128
    octets = nrows5 // 8
    big_o = -(-octets // NUM_WORKERS)
    n_big = octets - (big_o - 1) * NUM_WORKERS
    rows = big_o * 8
    trows = 64
    mesh = plsc.VectorSubcoreMesh(core_axis_name="c", subcore_axis_name="s")

    goff = [(i - (M_S - 1) / 2.0) / float(M_S) for i in range(M_S)]

    @functools.partial(
        pl.kernel,
        mesh=mesh,
        out_type=jax.ShapeDtypeStruct((NUM_WORKERS, 2 * K), jnp.float32),
        scratch_types=[
            pltpu.VMEM((rows, 128), jnp.float32),
            pltpu.VMEM((rows, 128), jnp.float32),
            pltpu.VMEM((rows, 128), jnp.float32),
            pltpu.VMEM((rows, 128), jnp.float32),
            pltpu.VMEM((64 * M_S * w_s + K,), jnp.float32),
            pltpu.VMEM((2 * K,), jnp.float32),
        ],
    )
    def greedy(q4_hbm, df_hbm, oy_hbm, ox_hbm, tb_hbm, out_hbm,
               q4_v, df_v, oy_v, ox_v, tb_v, acc_v):
        wid = lax.axis_index("s") * 2 + lax.axis_index("c")
        o0 = big_o * wid - jnp.maximum(wid - n_big, 0)
        nblk = 64 * (big_o - jnp.minimum(jnp.maximum(wid - n_big + 1, 0), 1))
        srow = 8 * o0
        srow_cl = jnp.minimum(srow, nrows5 - rows)
        rowoff = srow - srow_cl
        g0 = 64 * o0
        lo = M_S * (g0 // w_s)
        start8 = jnp.minimum(8 * (lo // 8), tb_rows - trows)
        pltpu.sync_copy(q4_hbm.at[pl.ds(srow_cl, rows)], q4_v)
        pltpu.sync_copy(df_hbm.at[pl.ds(srow_cl, rows)], df_v)
        pltpu.sync_copy(oy_hbm.at[pl.ds(srow_cl, rows)], oy_v)
        pltpu.sync_copy(ox_hbm.at[pl.ds(srow_cl, rows)], ox_v)
        w_full = M_S * w_s
        pltpu.sync_copy(tb_hbm.at[pl.ds(start8 * w_full, trows * w_full)],
                        tb_v.at[pl.ds(0, trows * w_full)])

        iota = lax.iota(jnp.int32, K)
        iota_f = iota.astype(jnp.float32)
        huge = jnp.float32(1e30)
        xor_idx = [jnp.bitwise_xor(iota, jnp.int32(s)) for s in (1, 2, 4, 8)]

        def _perm(x, idx):
            return lax.gather(
                x, idx[:, None],
                dimension_numbers=lax.GatherDimensionNumbers(
                    offset_dims=(), collapsed_slice_dims=(0,),
                    start_index_map=(0,)),
                slice_sizes=(1,),
                mode=lax.GatherScatterMode.PROMISE_IN_BOUNDS)

        def _allmin(x):
            for idx in xor_idx:
                x = jnp.minimum(x, _perm(x, idx))
            return x

        def one_block(b):
            r = rowoff + b // 8
            c = (b % 8) * K
            qv = q4_v[r, pl.ds(c, K)]
            dfv = df_v[r, pl.ds(c, K)]
            oyv = oy_v[r, pl.ds(c, K)]
            oxv = ox_v[r, pl.ds(c, K)]
            g = g0 + b
            ws = g % w_s
            rb = M_S * (g // w_s) - start8
            base8 = rb * w_full + 8 * (ws // 2)
            o4 = (ws % 2).astype(jnp.float32)
            o4c = 1.0 - o4
            trow = [tb_v[pl.ds(base8 + w_full * jy, K)] for jy in range(M_S)]
            tj = [trow[j // M_S][j % M_S] * o4c
                  + trow[j // M_S][4 + (j % M_S)] * o4
                  for j in range(K)]
            dys = [jnp.abs(oyv - g_) for g_ in goff]
            dxs = [jnp.abs(oxv - g_) for g_ in goff]
            dyqs = [dy * qv for dy in dys]
            dxqs = [dx * qv for dx in dxs]
            used = jnp.zeros((K,), jnp.float32)
            bc = jnp.zeros((K,), jnp.float32)
            ts = jnp.float32(0.0)
            for j in range(K):
                jy, jx = j // M_S, j % M_S
                keyv = dyqs[jy] + dxqs[jx] + used * huge
                minv = _allmin(keyv)
                sel = (1.0 - jnp.sign(keyv - minv)) * tj[j]
                used = used + sel
                bc = bc + sel * (dfv + 10.0 * (dys[jy] + dxs[jx]))
                ts = ts + tj[j]
            return bc, ts

        def block_body(i, carry):
            accv, nps = carry
            bc, ts = one_block(i)
            return accv + bc, nps + ts

        accv, nps = lax.fori_loop(
            0, nblk, block_body,
            (jnp.zeros((K,), jnp.float32), jnp.float32(0.0)))
        acc_v[pl.ds(0, K)] = accv
        acc_v[pl.ds(K, K)] = (1.0 - jnp.abs(jnp.sign(iota_f))) * nps
        pltpu.sync_copy(acc_v, out_hbm.at[wid])

    return greedy


def kernel(pred_scores, pred_offsets, H, W, M, target):
    B, HW, _ = pred_scores.shape
    n_blocks = B * HW
    h_s = target.shape[1] // M_S
    w_s = target.shape[2] // M_S

    tb = (target > 0).astype(jnp.float32).reshape(-1)
    s = pred_scores.reshape(n_blocks, K)
    oy = pred_offsets[..., 0].reshape(n_blocks, K)
    ox = pred_offsets[..., 1].reshape(n_blocks, K)

    ncols = (n_blocks * K) // 128
    grid = 5
    q4, df, s0p = _dense_stage(s.reshape(ncols, 128), grid)

    partials = _make_greedy_kernel(n_blocks, w_s, B * target.shape[1])(
        q4, df, oy.reshape(ncols, 128), ox.reshape(ncols, 128), tb)

    npos = jnp.maximum(partials[:, K:].sum(), 1.0)
    return (s0p.sum() + partials[:, :K].sum()) / npos
